# baseline (speedup 1.0000x reference)
import functools

import jax
import jax.numpy as jnp
from jax import lax
from jax.experimental import pallas as pl
from jax.experimental.pallas import tpu as pltpu
from jax.experimental.pallas import tpu_sc as plsc

_N = 10000
_B = 4096
_D = 128
_R = 256
_NB = _B // _R
_C = 4
_RC = _R // _C

_SC_NC = 2
_SC_NS = 16
_SC_NW = _SC_NC * _SC_NS
_B_PER_W = _B // _SC_NW


def _tc_body(ulist_smem, nu_smem, w_hbm, raw_ref, wt_ref, b_ref, ulist3d_ref,
             out_ref, rows_a, rows_b, rows_c, sem_a, sem_b, sem_c):
    i = pl.program_id(0)
    nu = nu_smem[0]

    def issue_chunk(blk, rows_ref, sem, c):
        @pl.when(blk * _R + c * _RC < nu)
        def _():
            for r in range(c * _RC, (c + 1) * _RC):
                node = ulist_smem[blk * _R + r]
                pltpu.make_async_copy(
                    w_hbm.at[node], rows_ref.at[r], sem
                ).start()

    def wait_chunks(rows_ref, sem):
        for c in range(_C):
            @pl.when(i * _R + c * _RC < nu)
            def _(c=c):
                pltpu.make_async_copy(
                    w_hbm.at[pl.ds(0, _RC)],
                    rows_ref.at[pl.ds(c * _RC, _RC)], sem
                ).wait()

    def compute_chunk(rows_ref, c):
        rows = rows_ref[pl.ds(c * _RC, _RC), :]
        nodes_col = ulist3d_ref[0, pl.ds(c * _RC, _RC), :]
        cols = jax.lax.broadcasted_iota(jnp.int32, (_RC, _N), 1)
        rows = jnp.where(cols == nodes_col, 0.0, rows)
        agg = jnp.dot(rows, raw_ref[...], preferred_element_type=jnp.float32,
                      precision=jax.lax.Precision.DEFAULT)
        out = jnp.dot(agg, wt_ref[...], preferred_element_type=jnp.float32)
        out_ref[pl.ds(c * _RC, _RC), :] = jnp.maximum(out + b_ref[...], 0.0)

    def step(cur_rows, cur_sem, nxt_rows, nxt_sem):
        wait_chunks(cur_rows, cur_sem)
        for c in range(_C):
            compute_chunk(cur_rows, c)

            @pl.when(i + 2 < _NB)
            def _(c=c):
                issue_chunk(i + 2, nxt_rows, nxt_sem, c)

    @pl.when(i == 0)
    def _():
        for c in range(_C):
            issue_chunk(0, rows_a, sem_a, c)
            issue_chunk(1, rows_b, sem_b, c)

    slot = jax.lax.rem(i, 3)

    @pl.when(slot == 0)
    def _():
        step(rows_a, sem_a, rows_c, sem_c)

    @pl.when(slot == 1)
    def _():
        step(rows_b, sem_b, rows_a, sem_a)

    @pl.when(slot == 2)
    def _():
        step(rows_c, sem_c, rows_b, sem_b)


@functools.partial(
    pl.kernel,
    out_type=jax.ShapeDtypeStruct((_B, _D), jnp.float32),
    mesh=plsc.VectorSubcoreMesh(core_axis_name="c", subcore_axis_name="s"),
    scratch_types=[
        pltpu.VMEM((_B_PER_W,), jnp.int32),
        pltpu.VMEM((_B_PER_W, _D), jnp.float32),
        pltpu.SemaphoreType.DMA,
    ],
)
def _sc_expand(inv_hbm, outu_hbm, out_hbm, idx_v, rows_v, sem):
    wid = lax.axis_index("s") * _SC_NC + lax.axis_index("c")
    base = wid * _B_PER_W
    pltpu.sync_copy(inv_hbm.at[pl.ds(base, _B_PER_W)], idx_v)
    pltpu.async_copy(outu_hbm.at[idx_v], rows_v, sem).wait()
    pltpu.sync_copy(rows_v, out_hbm.at[pl.ds(base, _B_PER_W)])


@jax.jit
def kernel(nodes_batch, raw_features, weighted_adj, W, b):
    nodes = nodes_batch.astype(jnp.int32)
    wt = W.T
    b2 = b.reshape(1, _D)

    present = jnp.zeros((_N,), jnp.int32).at[nodes].set(1)
    slots = jnp.cumsum(present, dtype=jnp.int32)
    nu_arr = slots[-1:]
    inv = jnp.take(slots, nodes) - 1
    vals = jnp.where(present == 1, jnp.arange(_N, dtype=jnp.int32), -1)
    ulist = jnp.zeros((_B,), jnp.int32).at[slots - 1].max(vals, mode="drop")

    grid_spec = pltpu.PrefetchScalarGridSpec(
        num_scalar_prefetch=2,
        grid=(_NB,),
        in_specs=[
            pl.BlockSpec(memory_space=pl.ANY),
            pl.BlockSpec((_N, _D), lambda i, ul, nu: (0, 0)),
            pl.BlockSpec((_D, _D), lambda i, ul, nu: (0, 0)),
            pl.BlockSpec((1, _D), lambda i, ul, nu: (0, 0)),
            pl.BlockSpec((1, _R, 1), lambda i, ul, nu: (i, 0, 0)),
        ],
        out_specs=pl.BlockSpec((_R, _D), lambda i, ul, nu: (i, 0)),
        scratch_shapes=[
            pltpu.VMEM((_R, _N), jnp.float32),
            pltpu.VMEM((_R, _N), jnp.float32),
            pltpu.VMEM((_R, _N), jnp.float32),
            pltpu.SemaphoreType.DMA,
            pltpu.SemaphoreType.DMA,
            pltpu.SemaphoreType.DMA,
        ],
    )
    out_u = pl.pallas_call(
        _tc_body,
        grid_spec=grid_spec,
        out_shape=jax.ShapeDtypeStruct((_B, _D), jnp.float32),
    )(ulist, nu_arr, weighted_adj, raw_features, wt, b2,
      ulist.reshape(_NB, _R, 1))

    return _sc_expand(inv, out_u)


# speedup vs baseline: 2.3412x; 2.3412x over previous
import functools

import jax
import jax.numpy as jnp
from jax.experimental import pallas as pl
from jax.experimental.pallas import tpu as pltpu

_N = 10000
_B = 4096
_D = 128
_R = 256
_NB = _B // _R
_PC = 2
_NBC = _NB // _PC
_C = 4
_RC = _R // _C


def _body(nodes_smem, w_hbm, raw_ref, wt_ref, b_ref, nodes3d_ref, out_ref,
          rows_a, rows_b, rows_c, sem_a, sem_b, sem_c):
    p = pl.program_id(0)
    j = pl.program_id(1)
    i = p * _NBC + j

    def issue_rows(blk, rows_ref, sem, lo, hi):
        for r in range(lo, hi):
            node = nodes_smem[blk * _R + r]
            pltpu.make_async_copy(
                w_hbm.at[node], rows_ref.at[r], sem
            ).start()

    def wait_block(rows_ref, sem):
        pltpu.make_async_copy(
            w_hbm.at[pl.ds(0, _R)], rows_ref, sem
        ).wait()

    def compute_chunk(rows_ref, c):
        rows = rows_ref[pl.ds(c * _RC, _RC), :]
        nodes_col = nodes3d_ref[0, pl.ds(c * _RC, _RC), :]
        cols = jax.lax.broadcasted_iota(jnp.int32, (_RC, _N), 1)
        rows = jnp.where(cols == nodes_col, 0.0, rows)
        agg = jnp.dot(rows, raw_ref[...], preferred_element_type=jnp.float32,
                      precision=jax.lax.Precision.DEFAULT)
        out = jnp.dot(agg, wt_ref[...], preferred_element_type=jnp.float32)
        out_ref[pl.ds(c * _RC, _RC), :] = jnp.maximum(out + b_ref[...], 0.0)

    def step(cur_rows, cur_sem, nxt_rows, nxt_sem):
        wait_block(cur_rows, cur_sem)
        for c in range(_C):
            compute_chunk(cur_rows, c)

            @pl.when(j + 2 < _NBC)
            def _(c=c):
                issue_rows(i + 2, nxt_rows, nxt_sem, c * _RC, (c + 1) * _RC)

    @pl.when(j == 0)
    def _():
        issue_rows(i, rows_a, sem_a, 0, _R)
        issue_rows(i + 1, rows_b, sem_b, 0, _R)

    slot = jax.lax.rem(j, 3)

    @pl.when(slot == 0)
    def _():
        step(rows_a, sem_a, rows_c, sem_c)

    @pl.when(slot == 1)
    def _():
        step(rows_b, sem_b, rows_a, sem_a)

    @pl.when(slot == 2)
    def _():
        step(rows_c, sem_c, rows_b, sem_b)


@jax.jit
def kernel(nodes_batch, raw_features, weighted_adj, W, b):
    nodes = nodes_batch.astype(jnp.int32)
    wt = W.T
    b2 = b.reshape(1, _D)

    grid_spec = pltpu.PrefetchScalarGridSpec(
        num_scalar_prefetch=1,
        grid=(_PC, _NBC),
        in_specs=[
            pl.BlockSpec(memory_space=pl.ANY),
            pl.BlockSpec((_N, _D), lambda p, j, ns: (0, 0)),
            pl.BlockSpec((_D, _D), lambda p, j, ns: (0, 0)),
            pl.BlockSpec((1, _D), lambda p, j, ns: (0, 0)),
            pl.BlockSpec((1, _R, 1), lambda p, j, ns: (p * _NBC + j, 0, 0)),
        ],
        out_specs=pl.BlockSpec((_R, _D), lambda p, j, ns: (p * _NBC + j, 0)),
        scratch_shapes=[
            pltpu.VMEM((_R, _N), jnp.float32),
            pltpu.VMEM((_R, _N), jnp.float32),
            pltpu.VMEM((_R, _N), jnp.float32),
            pltpu.SemaphoreType.DMA,
            pltpu.SemaphoreType.DMA,
            pltpu.SemaphoreType.DMA,
        ],
    )
    return pl.pallas_call(
        _body,
        grid_spec=grid_spec,
        out_shape=jax.ShapeDtypeStruct((_B, _D), jnp.float32),
        compiler_params=pltpu.CompilerParams(
            dimension_semantics=("parallel", "arbitrary")),
    )(nodes, weighted_adj, raw_features, wt, b2,
      nodes.reshape(_NB, _R, 1))


# speedup vs baseline: 2.5525x; 1.0902x over previous
import functools

import jax
import jax.numpy as jnp
from jax.experimental import pallas as pl
from jax.experimental.pallas import tpu as pltpu

_N = 10000
_B = 4096
_D = 128
_R = 256
_NB = _B // _R
_C = 4
_RC = _R // _C


def _body(nodes_smem, w_hbm, raw_ref, wt_ref, b_ref, nodes3d_ref, out_ref,
          rows_a, rows_b, rows_c, sem_a, sem_b, sem_c):
    i = pl.program_id(0)

    def issue_rows(blk, rows_ref, sem, lo, hi):
        for r in range(lo, hi):
            node = nodes_smem[blk * _R + r]
            pltpu.make_async_copy(
                w_hbm.at[node], rows_ref.at[r], sem
            ).start()

    def wait_block(rows_ref, sem):
        pltpu.make_async_copy(
            w_hbm.at[pl.ds(0, _R)], rows_ref, sem
        ).wait()

    def compute_chunk(rows_ref, c):
        rows = rows_ref[pl.ds(c * _RC, _RC), :]
        nodes_col = nodes3d_ref[0, pl.ds(c * _RC, _RC), :]
        cols = jax.lax.broadcasted_iota(jnp.int32, (_RC, _N), 1)
        rows = jnp.where(cols == nodes_col, 0.0, rows)
        agg = jnp.dot(rows, raw_ref[...], preferred_element_type=jnp.float32,
                      precision=jax.lax.Precision.DEFAULT)
        out = jnp.dot(agg, wt_ref[...], preferred_element_type=jnp.float32)
        out_ref[pl.ds(c * _RC, _RC), :] = jnp.maximum(out + b_ref[...], 0.0)

    def step(cur_rows, cur_sem, nxt_rows, nxt_sem):
        wait_block(cur_rows, cur_sem)
        for c in range(_C):
            compute_chunk(cur_rows, c)

            @pl.when(i + 2 < _NB)
            def _(c=c):
                issue_rows(i + 2, nxt_rows, nxt_sem, c * _RC, (c + 1) * _RC)

    @pl.when(i == 0)
    def _():
        issue_rows(0, rows_a, sem_a, 0, _R)
        issue_rows(1, rows_b, sem_b, 0, _R)

    slot = jax.lax.rem(i, 3)

    @pl.when(slot == 0)
    def _():
        step(rows_a, sem_a, rows_c, sem_c)

    @pl.when(slot == 1)
    def _():
        step(rows_b, sem_b, rows_a, sem_a)

    @pl.when(slot == 2)
    def _():
        step(rows_c, sem_c, rows_b, sem_b)


@jax.jit
def kernel(nodes_batch, raw_features, weighted_adj, W, b):
    nodes = nodes_batch.astype(jnp.int32)
    wt = W.T
    b2 = b.reshape(1, _D)

    grid_spec = pltpu.PrefetchScalarGridSpec(
        num_scalar_prefetch=1,
        grid=(_NB,),
        in_specs=[
            pl.BlockSpec(memory_space=pl.ANY),
            pl.BlockSpec((_N, _D), lambda i, ns: (0, 0)),
            pl.BlockSpec((_D, _D), lambda i, ns: (0, 0)),
            pl.BlockSpec((1, _D), lambda i, ns: (0, 0)),
            pl.BlockSpec((1, _R, 1), lambda i, ns: (i, 0, 0)),
        ],
        out_specs=pl.BlockSpec((_R, _D), lambda i, ns: (i, 0)),
        scratch_shapes=[
            pltpu.VMEM((_R, _N), jnp.float32),
            pltpu.VMEM((_R, _N), jnp.float32),
            pltpu.VMEM((_R, _N), jnp.float32),
            pltpu.SemaphoreType.DMA,
            pltpu.SemaphoreType.DMA,
            pltpu.SemaphoreType.DMA,
        ],
    )
    return pl.pallas_call(
        _body,
        grid_spec=grid_spec,
        out_shape=jax.ShapeDtypeStruct((_B, _D), jnp.float32),
    )(nodes, weighted_adj, raw_features, wt, b2,
      nodes.reshape(_NB, _R, 1))


# speedup vs baseline: 2.6454x; 1.0364x over previous
import functools

import jax
import jax.numpy as jnp
from jax.experimental import pallas as pl
from jax.experimental.pallas import tpu as pltpu

_N = 10000
_B = 4096
_D = 128
_R = 128
_NB = _B // _R
_C = 2
_RC = _R // _C


def _body(nodes_smem, w_hbm, raw_ref, wt_ref, b_ref, nodes3d_ref, out_ref,
          rows_a, rows_b, rows_c, sem_a, sem_b, sem_c):
    i = pl.program_id(0)

    def issue_rows(blk, rows_ref, sem, lo, hi):
        for r in range(lo, hi):
            node = nodes_smem[blk * _R + r]
            pltpu.make_async_copy(
                w_hbm.at[node], rows_ref.at[r], sem
            ).start()

    def wait_block(rows_ref, sem):
        pltpu.make_async_copy(
            w_hbm.at[pl.ds(0, _R)], rows_ref, sem
        ).wait()

    def compute_chunk(rows_ref, c):
        rows = rows_ref[pl.ds(c * _RC, _RC), :]
        nodes_col = nodes3d_ref[0, pl.ds(c * _RC, _RC), :]
        cols = jax.lax.broadcasted_iota(jnp.int32, (_RC, _N), 1)
        rows = jnp.where(cols == nodes_col, 0.0, rows)
        agg = jnp.dot(rows, raw_ref[...], preferred_element_type=jnp.float32,
                      precision=jax.lax.Precision.DEFAULT)
        out = jnp.dot(agg, wt_ref[...], preferred_element_type=jnp.float32)
        out_ref[pl.ds(c * _RC, _RC), :] = jnp.maximum(out + b_ref[...], 0.0)

    def step(cur_rows, cur_sem, nxt_rows, nxt_sem):
        wait_block(cur_rows, cur_sem)
        for c in range(_C):
            compute_chunk(cur_rows, c)

            @pl.when(i + 2 < _NB)
            def _(c=c):
                issue_rows(i + 2, nxt_rows, nxt_sem, c * _RC, (c + 1) * _RC)

    @pl.when(i == 0)
    def _():
        issue_rows(0, rows_a, sem_a, 0, _R)
        issue_rows(1, rows_b, sem_b, 0, _R)

    slot = jax.lax.rem(i, 3)

    @pl.when(slot == 0)
    def _():
        step(rows_a, sem_a, rows_c, sem_c)

    @pl.when(slot == 1)
    def _():
        step(rows_b, sem_b, rows_a, sem_a)

    @pl.when(slot == 2)
    def _():
        step(rows_c, sem_c, rows_b, sem_b)


@jax.jit
def kernel(nodes_batch, raw_features, weighted_adj, W, b):
    nodes = nodes_batch.astype(jnp.int32)
    wt = W.T
    b2 = b.reshape(1, _D)

    grid_spec = pltpu.PrefetchScalarGridSpec(
        num_scalar_prefetch=1,
        grid=(_NB,),
        in_specs=[
            pl.BlockSpec(memory_space=pl.ANY),
            pl.BlockSpec((_N, _D), lambda i, ns: (0, 0)),
            pl.BlockSpec((_D, _D), lambda i, ns: (0, 0)),
            pl.BlockSpec((1, _D), lambda i, ns: (0, 0)),
            pl.BlockSpec((1, _R, 1), lambda i, ns: (i, 0, 0)),
        ],
        out_specs=pl.BlockSpec((_R, _D), lambda i, ns: (i, 0)),
        scratch_shapes=[
            pltpu.VMEM((_R, _N), jnp.float32),
            pltpu.VMEM((_R, _N), jnp.float32),
            pltpu.VMEM((_R, _N), jnp.float32),
            pltpu.SemaphoreType.DMA,
            pltpu.SemaphoreType.DMA,
            pltpu.SemaphoreType.DMA,
        ],
    )
    return pl.pallas_call(
        _body,
        grid_spec=grid_spec,
        out_shape=jax.ShapeDtypeStruct((_B, _D), jnp.float32),
    )(nodes, weighted_adj, raw_features, wt, b2,
      nodes.reshape(_NB, _R, 1))
